# baseline (speedup 1.0000x reference)
import functools

import jax
import jax.numpy as jnp
from jax import lax
from jax.experimental import pallas as pl
from jax.experimental.pallas import tpu as pltpu
from jax.experimental.pallas import tpu_sc as plsc

CTX_DIM = 512
CONTEXT_LEN = 77
N_CTX = 16
N_GATHER = CONTEXT_LEN - N_CTX


def _sc_body(gidx_hbm, table_hbm, ctx_hbm, out_hbm, idx_v, rows_v, sem):
    nc = plsc.get_sparse_core_info().num_cores
    wid = lax.axis_index("s") * nc + lax.axis_index("c")

    pltpu.sync_copy(gidx_hbm, idx_v)

    @pl.when(wid < 31)
    def _():
        pltpu.async_copy(table_hbm.at[idx_v.at[wid]], rows_v, sem).wait()

    @pl.when(wid < 30)
    def _():
        pltpu.sync_copy(rows_v, out_hbm.at[pl.ds(N_CTX + 1 + 2 * wid, 2)])

    @pl.when(wid == 30)
    def _():
        pltpu.sync_copy(rows_v.at[pl.ds(0, 1)], out_hbm.at[pl.ds(0, 1)])

    @pl.when(wid == 31)
    def _():
        pltpu.sync_copy(ctx_hbm, out_hbm.at[pl.ds(1, N_CTX)])


@jax.jit
def _sc_call(gidx, token_embedding, ctx):
    mesh = plsc.VectorSubcoreMesh(core_axis_name="c", subcore_axis_name="s")
    return pl.kernel(
        _sc_body,
        out_type=jax.ShapeDtypeStruct((CONTEXT_LEN, CTX_DIM), jnp.float32),
        mesh=mesh,
        scratch_types=[
            pltpu.VMEM((32, 2), jnp.int32),
            pltpu.VMEM((2, CTX_DIM), jnp.float32),
            pltpu.SemaphoreType.DMA,
        ],
        compiler_params=pltpu.CompilerParams(use_tc_tiling_on_sc=False),
    )(gidx, token_embedding, ctx)


def kernel(tokenized_prompts, token_embedding, ctx):
    tok = tokenized_prompts[0].astype(jnp.int32)
    pairs = tok[1:N_GATHER].reshape(30, 2)
    sos = jnp.stack([tok[0], tok[0]])[None]
    pad = jnp.zeros((1, 2), jnp.int32)
    gidx = jnp.concatenate([pairs, sos, pad], axis=0)
    return _sc_call(gidx, token_embedding, ctx)


# speedup vs baseline: 4.1670x; 4.1670x over previous
import numpy as np
import jax
import jax.numpy as jnp
from jax import lax
from jax.experimental import pallas as pl
from jax.experimental.pallas import tpu as pltpu
from jax.experimental.pallas import tpu_sc as plsc

CTX_DIM = 512
CONTEXT_LEN = 77
N_CTX = 16
N_TOK = CONTEXT_LEN - N_CTX
N_GATHER = 64


def _sc_body(tok_hbm, table_hbm, ctx_hbm, sidx_hbm, cidx_hbm, out_hbm,
             tokv, gidxv, sidxv, cidxv, gv, cv, sem, sem2):
    nc = plsc.get_sparse_core_info().num_cores
    wid = lax.axis_index("s") * nc + lax.axis_index("c")

    @pl.when(wid == 0)
    def _():
        cp1 = pltpu.async_copy(tok_hbm, tokv, sem)
        cp2 = pltpu.async_copy(sidx_hbm, sidxv, sem2)
        cp1.wait()
        cp2.wait()
        zeros = jnp.zeros((16,), jnp.int32)
        for i in range(N_GATHER // 16):
            lane = jax.lax.iota(jnp.int32, 16) + 16 * i
            col = jnp.minimum(lane, N_TOK - 1)
            gidxv[pl.ds(16 * i, 16)] = plsc.load_gather(tokv, [zeros, col])
        pltpu.async_copy(table_hbm.at[gidxv], gv, sem).wait()
        pltpu.async_copy(gv, out_hbm.at[sidxv], sem).wait()

    @pl.when(wid == 1)
    def _():
        cp1 = pltpu.async_copy(cidx_hbm, cidxv, sem)
        cp2 = pltpu.async_copy(ctx_hbm, cv, sem2)
        cp1.wait()
        cp2.wait()
        pltpu.async_copy(cv, out_hbm.at[cidxv], sem).wait()


@jax.jit
def _sc_call(tok, token_embedding, ctx):
    sidx = np.concatenate(
        [[0], np.arange(N_CTX + 1, CONTEXT_LEN),
         [CONTEXT_LEN - 1] * (N_GATHER - N_TOK)]).astype(np.int32)
    cidx = np.arange(1, N_CTX + 1, dtype=np.int32)
    mesh = plsc.VectorSubcoreMesh(core_axis_name="c", subcore_axis_name="s")
    return pl.kernel(
        _sc_body,
        out_type=jax.ShapeDtypeStruct((CONTEXT_LEN, CTX_DIM), jnp.float32),
        mesh=mesh,
        scratch_types=[
            pltpu.VMEM((1, CONTEXT_LEN), jnp.int32),
            pltpu.VMEM((N_GATHER,), jnp.int32),
            pltpu.VMEM((N_GATHER,), jnp.int32),
            pltpu.VMEM((N_CTX,), jnp.int32),
            pltpu.VMEM((N_GATHER, CTX_DIM), jnp.float32),
            pltpu.VMEM((N_CTX, CTX_DIM), jnp.float32),
            pltpu.SemaphoreType.DMA,
            pltpu.SemaphoreType.DMA,
        ],
        compiler_params=pltpu.CompilerParams(needs_layout_passes=False),
    )(tok, token_embedding, ctx, jnp.asarray(sidx), jnp.asarray(cidx))


def kernel(tokenized_prompts, token_embedding, ctx):
    return _sc_call(tokenized_prompts.astype(jnp.int32), token_embedding, ctx)


# speedup vs baseline: 4.4735x; 1.0735x over previous
import jax
import jax.numpy as jnp
from jax import lax
from jax.experimental import pallas as pl
from jax.experimental.pallas import tpu as pltpu
from jax.experimental.pallas import tpu_sc as plsc

CTX_DIM = 512
CONTEXT_LEN = 77
N_CTX = 16
N_TOK = CONTEXT_LEN - N_CTX
N_GATHER = 64


def _sc_body(tok_hbm, table_hbm, ctx_hbm, out_hbm,
             tokv, gidxv, sidxv, cidxv, gv, cv, sem):
    wid = lax.axis_index("s")

    @pl.when(wid == 0)
    def _():
        cp_tok = pltpu.async_copy(tok_hbm, tokv, sem)
        for i in range(N_GATHER // 16):
            lane = lax.iota(jnp.int32, 16) + 16 * i
            row = jnp.where(lane == 0, 0,
                            jnp.minimum(lane + N_CTX, CONTEXT_LEN - 1))
            sidxv[pl.ds(16 * i, 16)] = row
        cp_tok.wait()
        zeros = jnp.zeros((16,), jnp.int32)
        for i in range(N_GATHER // 16):
            lane = lax.iota(jnp.int32, 16) + 16 * i
            col = jnp.minimum(lane, N_TOK - 1)
            gidxv[pl.ds(16 * i, 16)] = plsc.load_gather(tokv, [zeros, col])
        pltpu.async_copy(table_hbm.at[gidxv], gv, sem).wait()
        pltpu.async_copy(gv, out_hbm.at[sidxv], sem).wait()

    @pl.when(wid == 1)
    def _():
        cp_ctx = pltpu.async_copy(ctx_hbm, cv, sem)
        cidxv[...] = lax.iota(jnp.int32, 16) + 1
        cp_ctx.wait()
        pltpu.async_copy(cv, out_hbm.at[cidxv], sem).wait()


@jax.jit
def _sc_call(tok, token_embedding, ctx):
    mesh = plsc.VectorSubcoreMesh(
        core_axis_name="c", subcore_axis_name="s", num_cores=1)
    return pl.kernel(
        _sc_body,
        out_type=jax.ShapeDtypeStruct((CONTEXT_LEN, CTX_DIM), jnp.float32),
        mesh=mesh,
        scratch_types=[
            pltpu.VMEM((1, CONTEXT_LEN), jnp.int32),
            pltpu.VMEM((N_GATHER,), jnp.int32),
            pltpu.VMEM((N_GATHER,), jnp.int32),
            pltpu.VMEM((N_CTX,), jnp.int32),
            pltpu.VMEM((N_GATHER, CTX_DIM), jnp.float32),
            pltpu.VMEM((N_CTX, CTX_DIM), jnp.float32),
            pltpu.SemaphoreType.DMA,
        ],
        compiler_params=pltpu.CompilerParams(needs_layout_passes=False),
    )(tok, token_embedding, ctx)


def kernel(tokenized_prompts, token_embedding, ctx):
    return _sc_call(tokenized_prompts.astype(jnp.int32), token_embedding, ctx)


# speedup vs baseline: 4.9122x; 1.0981x over previous
import jax
import jax.numpy as jnp
from jax import lax
from jax.experimental import pallas as pl
from jax.experimental.pallas import tpu as pltpu
from jax.experimental.pallas import tpu_sc as plsc

CTX_DIM = 512
CONTEXT_LEN = 77
N_CTX = 16
N_TOK = CONTEXT_LEN - N_CTX
N_WORK = 4


def _sc_body(tok_hbm, table_hbm, ctx_hbm, out_hbm,
             tokv, gidxv, sidxv, cidxv, gv, cv, sem):
    wid = lax.axis_index("s")

    @pl.when(wid < N_WORK)
    def _():
        cp_tok = pltpu.async_copy(tok_hbm, tokv, sem)
        slot = lax.iota(jnp.int32, 16) + 16 * wid
        row = jnp.where(slot == 0, 0,
                        jnp.minimum(slot + N_CTX, CONTEXT_LEN - 1))
        sidxv[...] = row
        cp_tok.wait()
        col = jnp.minimum(slot, N_TOK - 1)
        gidxv[...] = plsc.load_gather(tokv, [jnp.zeros((16,), jnp.int32), col])
        pltpu.async_copy(table_hbm.at[gidxv], gv, sem).wait()
        pltpu.async_copy(gv, out_hbm.at[sidxv], sem).wait()

    @pl.when(wid == N_WORK)
    def _():
        cp_ctx = pltpu.async_copy(ctx_hbm, cv, sem)
        cidxv[...] = lax.iota(jnp.int32, 16) + 1
        cp_ctx.wait()
        pltpu.async_copy(cv, out_hbm.at[cidxv], sem).wait()


@jax.jit
def _sc_call(tok, token_embedding, ctx):
    mesh = plsc.VectorSubcoreMesh(
        core_axis_name="c", subcore_axis_name="s", num_cores=1)
    return pl.kernel(
        _sc_body,
        out_type=jax.ShapeDtypeStruct((CONTEXT_LEN, CTX_DIM), jnp.float32),
        mesh=mesh,
        scratch_types=[
            pltpu.VMEM((1, CONTEXT_LEN), jnp.int32),
            pltpu.VMEM((16,), jnp.int32),
            pltpu.VMEM((16,), jnp.int32),
            pltpu.VMEM((N_CTX,), jnp.int32),
            pltpu.VMEM((16, CTX_DIM), jnp.float32),
            pltpu.VMEM((N_CTX, CTX_DIM), jnp.float32),
            pltpu.SemaphoreType.DMA,
        ],
        compiler_params=pltpu.CompilerParams(needs_layout_passes=False),
    )(tok, token_embedding, ctx)


def kernel(tokenized_prompts, token_embedding, ctx):
    return _sc_call(tokenized_prompts.astype(jnp.int32), token_embedding, ctx)
